# baseline (speedup 1.0000x reference)
import jax
import jax.numpy as jnp
from jax.experimental import pallas as pl

_SIZE_IN = 4096
_SIZE_OUT = 4096
_BLOCK_J = 512


def _spmm_kernel(x_ref, w_ref, o_ref):
    o_ref[...] = jax.lax.dot_general(
        w_ref[...], x_ref[...],
        dimension_numbers=(((0,), (0,)), ((), ())),
        preferred_element_type=jnp.float32,
    )


def kernel(input, W):
    size_in, cols = input.shape
    size_out = W.shape[1]
    grid = (size_out // _BLOCK_J,)
    return pl.pallas_call(
        _spmm_kernel,
        grid=grid,
        in_specs=[
            pl.BlockSpec((size_in, cols), lambda j: (0, 0)),
            pl.BlockSpec((size_in, _BLOCK_J), lambda j: (0, j)),
        ],
        out_specs=pl.BlockSpec((_BLOCK_J, cols), lambda j: (j, 0)),
        out_shape=jax.ShapeDtypeStruct((size_out, cols), jnp.float32),
    )(input, W)


# speedup vs baseline: 1.0563x; 1.0563x over previous
import jax
import jax.numpy as jnp
from jax.experimental import pallas as pl
from jax.experimental.pallas import tpu as pltpu

_BLOCK_J = 512


def _spmm_kernel(x_ref, w_ref, o_ref, xt_ref):
    @pl.when(pl.program_id(0) == 0)
    def _():
        xt_ref[...] = x_ref[...].T

    acc = jax.lax.dot_general(
        xt_ref[...], w_ref[...],
        dimension_numbers=(((1,), (0,)), ((), ())),
        preferred_element_type=jnp.float32,
    )
    o_ref[...] = acc.T


def kernel(input, W):
    size_in, cols = input.shape
    size_out = W.shape[1]
    grid = (size_out // _BLOCK_J,)
    return pl.pallas_call(
        _spmm_kernel,
        grid=grid,
        in_specs=[
            pl.BlockSpec((size_in, cols), lambda j: (0, 0)),
            pl.BlockSpec((size_in, _BLOCK_J), lambda j: (0, j)),
        ],
        out_specs=pl.BlockSpec((_BLOCK_J, cols), lambda j: (j, 0)),
        out_shape=jax.ShapeDtypeStruct((size_out, cols), jnp.float32),
        scratch_shapes=[pltpu.VMEM((cols, size_in), jnp.float32)],
    )(input, W)


# speedup vs baseline: 1.0610x; 1.0044x over previous
import jax
import jax.numpy as jnp
from jax.experimental import pallas as pl
from jax.experimental.pallas import tpu as pltpu

_BLOCK_I = 512


def _spmm_kernel(x_ref, w_ref, o_ref, xt_ref, acc_ref):
    i = pl.program_id(0)
    nblk = pl.num_programs(0)

    @pl.when(i == 0)
    def _():
        xt_ref[...] = x_ref[...].T

    part = jax.lax.dot_general(
        xt_ref[:, pl.ds(i * _BLOCK_I, _BLOCK_I)], w_ref[...],
        dimension_numbers=(((1,), (0,)), ((), ())),
        preferred_element_type=jnp.float32,
    )

    @pl.when(i == 0)
    def _():
        acc_ref[...] = part

    @pl.when(i > 0)
    def _():
        acc_ref[...] += part

    @pl.when(i == nblk - 1)
    def _():
        o_ref[...] = acc_ref[...].T


def kernel(input, W):
    size_in, cols = input.shape
    size_out = W.shape[1]
    grid = (size_in // _BLOCK_I,)
    return pl.pallas_call(
        _spmm_kernel,
        grid=grid,
        in_specs=[
            pl.BlockSpec((size_in, cols), lambda i: (0, 0)),
            pl.BlockSpec((_BLOCK_I, size_out), lambda i: (i, 0)),
        ],
        out_specs=pl.BlockSpec((size_out, cols), lambda i: (0, 0)),
        out_shape=jax.ShapeDtypeStruct((size_out, cols), jnp.float32),
        scratch_shapes=[
            pltpu.VMEM((cols, size_in), jnp.float32),
            pltpu.VMEM((cols, size_out), jnp.float32),
        ],
    )(input, W)
